# baseline (speedup 1.0000x reference)
import functools

import jax
import jax.numpy as jnp
from jax import lax
from jax.experimental import pallas as pl
from jax.experimental.pallas import tpu as pltpu, tpu_sc as plsc

N_FEAT = 2000
N_CELL = 8000
E = 320000
H = 128
OUT = 128

_NS = 16
_EPT0 = 19968
_EPT15 = E - 15 * _EPT0
_EBUF = _EPT15
_CH = 786432
_NPASS = 21
_MPAD = _NPASS * _CH
_WPT = _CH // _NS
_SROUND = 1536
_ZB = 2048
_HB = 8192
_DUMP = _CH


def _leaky(x):
    return jnp.where(x >= 0, x, 0.01 * x)


def _ln(x, g, b, eps=1e-5):
    mu = jnp.mean(x, axis=-1, keepdims=True)
    var = jnp.mean((x - mu) ** 2, axis=-1, keepdims=True)
    return (x - mu) / jnp.sqrt(var + eps) * g + b


def _init_feat_body(ef_ref, w_ref, b_ref, g_ref, beta_ref, out_ref):
    x = _leaky(ef_ref[...])
    x = jnp.maximum(jnp.dot(x, w_ref[...], preferred_element_type=jnp.float32)
                    + b_ref[...], 0.0)
    out_ref[...] = _ln(x, g_ref[...], beta_ref[...])


def _init_cell_body(ec_ref, bf_ref, we_ref, be_ref, w_ref, b_ref, g_ref,
                    beta_ref, out_ref):
    extra = _leaky(jnp.dot(bf_ref[...], we_ref[...],
                           preferred_element_type=jnp.float32) + be_ref[...])
    x = _leaky(ec_ref[...]) + extra
    x = jnp.maximum(jnp.dot(x, w_ref[...], preferred_element_type=jnp.float32)
                    + b_ref[...], 0.0)
    out_ref[...] = _ln(x, g_ref[...], beta_ref[...])


def _conv_body(a_ref, hsrc_ref, hdst_ref, invdeg_ref, ws_ref, wn_ref, b_ref,
               g_ref, beta_ref, out_ref):
    s = jnp.dot(a_ref[...], hsrc_ref[...], preferred_element_type=jnp.float32)
    h_neigh = s * invdeg_ref[...]
    new = (jnp.dot(hdst_ref[...], ws_ref[...], preferred_element_type=jnp.float32)
           + jnp.dot(h_neigh, wn_ref[...], preferred_element_type=jnp.float32)
           + b_ref[...])
    out_ref[...] = jnp.maximum(_ln(new, g_ref[...], beta_ref[...]), 0.0)


def _readout_body(h1_ref, h2_ref, w0_ref, b0_ref, w1_ref, b1_ref, ca_ref,
                  cb_ref, out_ref):
    h = jnp.concatenate([h1_ref[...], h2_ref[...]], axis=1)
    h = jnp.maximum(jnp.dot(h, w0_ref[...], preferred_element_type=jnp.float32)
                    + b0_ref[...], 0.0)
    o = jnp.dot(h, w1_ref[...], preferred_element_type=jnp.float32) + b1_ref[...]
    out_ref[...] = o * ca_ref[...] + cb_ref[...]


def _full(shape):
    return pl.BlockSpec(shape, lambda *_: tuple(0 for _ in shape))


def _init_feat(ef, w, b, g, beta):
    return pl.pallas_call(
        _init_feat_body,
        out_shape=jax.ShapeDtypeStruct((N_FEAT, H), jnp.float32),
    )(ef, w, b[None, :], g[None, :], beta[None, :])


def _init_cell(ec, bf, we, be, w, b, g, beta):
    return pl.pallas_call(
        _init_cell_body,
        out_shape=jax.ShapeDtypeStruct((N_CELL, H), jnp.float32),
    )(ec, bf, we, be[None, :], w, b[None, :], g[None, :], beta[None, :])


def _conv(a, h_src, h_dst, inv_deg, ws, wn, b, g, beta, bm):
    n_dst, n_src = a.shape
    grid = (n_dst // bm,)
    return pl.pallas_call(
        _conv_body,
        grid=grid,
        in_specs=[
            pl.BlockSpec((bm, n_src), lambda i: (i, 0)),
            pl.BlockSpec((n_src, H), lambda i: (0, 0)),
            pl.BlockSpec((bm, H), lambda i: (i, 0)),
            pl.BlockSpec((bm, 1), lambda i: (i, 0)),
            pl.BlockSpec((H, H), lambda i: (0, 0)),
            pl.BlockSpec((H, H), lambda i: (0, 0)),
            pl.BlockSpec((1, H), lambda i: (0, 0)),
            pl.BlockSpec((1, H), lambda i: (0, 0)),
            pl.BlockSpec((1, H), lambda i: (0, 0)),
        ],
        out_specs=pl.BlockSpec((bm, H), lambda i: (i, 0)),
        out_shape=jax.ShapeDtypeStruct((n_dst, H), jnp.float32),
    )(a, h_src, h_dst, inv_deg, ws, wn, b[None, :], g[None, :], beta[None, :])


def _readout(h1, h2, w0, b0, w1, b1, ca, cb):
    return pl.pallas_call(
        _readout_body,
        out_shape=jax.ShapeDtypeStruct((N_CELL, OUT), jnp.float32),
    )(h1, h2, w0, b0[None, :], w1, b1[None, :], ca[None, :], cb[None, :])


def _densify_body(src_hbm, dst_hbm, ews_hbm, zeros_hbm,
                  af_hbm, ac_hbm, degc_hbm, degf_hbm,
                  src_v, dst_v, ew_v, binidx, binval, hist, zb,
                  cnt16, start16, cursor, stg, drain, acc, tmp,
                  chunk, hstage, sem):
    c = lax.axis_index("c")
    t = lax.axis_index("s")
    lane = lax.iota(jnp.int32, 16)
    is_f2c = c == 0
    stride = jnp.where(is_f2c, N_FEAT, N_CELL)
    last = t == _NS - 1
    base_e = t * _EPT0

    zi = jnp.broadcast_to(jnp.int32(0), (16,))
    zf = jnp.broadcast_to(jnp.float32(0.0), (16,))
    ones_i = jnp.broadcast_to(jnp.int32(1), (16,))
    onef = jnp.broadcast_to(jnp.float32(1.0), (16,))
    cnt16[pl.ds(0, 16)] = zi
    cnt16[pl.ds(16, 16)] = zi

    def zset(i, _):
        zb[pl.ds(i * 16, 16)] = zf
        return 0
    lax.fori_loop(0, _ZB // 16, zset, 0)

    def hz(i, _):
        hist[pl.ds(i * 16, 16)] = zf
        return 0
    lax.fori_loop(0, _HB // 16, hz, 0)

    def edge_vec(i):
        o = i * 16
        s16 = src_v[pl.ds(o, 16)]
        d16 = dst_v[pl.ds(o, 16)]
        row = jnp.where(is_f2c, d16, s16)
        flat = row * stride + jnp.where(is_f2c, s16, d16)
        return row, flat

    def count_step(i, _):
        row, flat = edge_vec(i)
        bucket = flat // _CH
        plsc.addupdate_scatter(cnt16, [bucket], ones_i)
        plsc.addupdate_scatter(hist, [row], onef)
        return 0

    def place_step(i, _):
        row, flat = edge_vec(i)
        w16 = ew_v[pl.ds(i * 16, 16)]
        bucket = flat // _CH
        rel = flat - bucket * _CH
        rank, _l = plsc.scan_count(bucket)
        base = plsc.load_gather(cursor, [bucket])
        dest = base + rank - 1
        plsc.store_scatter(binidx, [dest], rel)
        plsc.store_scatter(binval, [dest], w16)
        plsc.addupdate_scatter(cursor, [bucket], ones_i)
        return 0

    def scan_edges(step, with_ew):
        def round_body(r, _):
            o = base_e + r * _SROUND
            pltpu.sync_copy(src_hbm.at[pl.ds(o, _SROUND)],
                            src_v.at[pl.ds(0, _SROUND)])
            pltpu.sync_copy(dst_hbm.at[pl.ds(o, _SROUND)],
                            dst_v.at[pl.ds(0, _SROUND)])
            if with_ew:
                pltpu.sync_copy(ews_hbm.at[pl.ds(c * E + o, _SROUND)],
                                ew_v.at[pl.ds(0, _SROUND)])
            lax.fori_loop(0, _SROUND // 16, step, 0)
            return 0
        lax.fori_loop(0, _EPT0 // _SROUND, round_body, 0)

        @pl.when(last)
        def _():
            o = base_e + (_EPT0 // _SROUND) * _SROUND
            pltpu.sync_copy(src_hbm.at[pl.ds(o, 512)],
                            src_v.at[pl.ds(0, 512)])
            pltpu.sync_copy(dst_hbm.at[pl.ds(o, 512)],
                            dst_v.at[pl.ds(0, 512)])
            if with_ew:
                pltpu.sync_copy(ews_hbm.at[pl.ds(c * E + o, 512)],
                                ew_v.at[pl.ds(0, 512)])
            lax.fori_loop(0, 512 // 16, step, 0)

    scan_edges(count_step, with_ew=False)

    c0 = cnt16[pl.ds(0, 16)]
    c1 = cnt16[pl.ds(16, 16)]
    st0 = plsc.cumsum(c0) - c0
    st1 = plsc.cumsum(c1) - c1 + jnp.sum(c0)
    start16[pl.ds(0, 16)] = st0
    start16[pl.ds(16, 16)] = st1
    cursor[pl.ds(0, 16)] = st0
    cursor[pl.ds(16, 16)] = st1

    scan_edges(place_step, with_ew=True)

    def pass_body(p, _):
        def zcp(j, _):
            pltpu.async_copy(zb, chunk.at[pl.ds(t * _WPT + j * _ZB, _ZB)],
                             sem)
            return 0
        lax.fori_loop(0, _WPT // _ZB, zcp, 0)

        @pl.when(last)
        def _():
            pltpu.async_copy(zb.at[pl.ds(0, 128)],
                             chunk.at[pl.ds(_CH, 128)], sem)

        def zdrain(j, _):
            pltpu.make_async_copy(zeros_hbm, zb, sem).wait()
            return 0
        lax.fori_loop(0, _WPT // _ZB, zdrain, 0)

        @pl.when(last)
        def _():
            pltpu.make_async_copy(zeros_hbm.at[pl.ds(0, 128)],
                                  zb.at[pl.ds(0, 128)], sem).wait()
        plsc.subcore_barrier()

        sv0 = start16[pl.ds(0, 16)]
        sv1 = start16[pl.ds(16, 16)]
        cv0 = cnt16[pl.ds(0, 16)]
        cv1 = cnt16[pl.ds(16, 16)]
        start_p = jnp.sum(jnp.where(lane == p, sv0, 0)
                          + jnp.where(lane == p - 16, sv1, 0))
        cnt_p = jnp.sum(jnp.where(lane == p, cv0, 0)
                        + jnp.where(lane == p - 16, cv1, 0))
        nsteps = (cnt_p + 15) >> 4

        def sc_body(k, _):
            @pl.when(k >= 8)
            def _():
                pltpu.make_async_copy(zeros_hbm.at[pl.ds(0, 16)], drain,
                                      sem).wait()
            off = start_p + k * 16
            idx16 = binidx[pl.ds(off, 16)]
            val16 = binval[pl.ds(off, 16)]
            m = (k * 16 + lane) < cnt_p
            idx16 = jnp.where(m, idx16, _DUMP)
            val16 = jnp.where(m, val16, 0.0)
            so = (k & 7) * 16
            stg[pl.ds(so, 16)] = val16
            pltpu.async_copy(stg.at[pl.ds(so, 16)], chunk.at[idx16], sem,
                             add=True)
            return 0
        lax.fori_loop(0, nsteps, sc_body, 0)

        def drain_body(j, _):
            pltpu.make_async_copy(zeros_hbm.at[pl.ds(0, 16)], drain,
                                  sem).wait()
            return 0
        lax.fori_loop(0, jnp.minimum(nsteps, 8), drain_body, 0)

        plsc.subcore_barrier()
        hbm_off = p * _CH + t * _WPT

        @pl.when(is_f2c)
        def _():
            pltpu.sync_copy(chunk.at[pl.ds(t * _WPT, _WPT)],
                            af_hbm.at[pl.ds(hbm_off, _WPT)])

        @pl.when(jnp.logical_not(is_f2c))
        def _():
            pltpu.sync_copy(chunk.at[pl.ds(t * _WPT, _WPT)],
                            ac_hbm.at[pl.ds(hbm_off, _WPT)])
        plsc.subcore_barrier()
        return 0
    lax.fori_loop(0, _NPASS, pass_body, 0)

    hs = _HB // _NS
    pltpu.sync_copy(hist, hstage.at[pl.ds(t * _HB, _HB)])
    plsc.subcore_barrier()
    pltpu.sync_copy(hstage.at[pl.ds(t * hs, hs)], acc)

    def red_body(j, _):
        pltpu.sync_copy(hstage.at[pl.ds(j * _HB + t * hs, hs)], tmp)

        def add_body(i, _):
            acc[pl.ds(i * 16, 16)] = acc[pl.ds(i * 16, 16)] + tmp[pl.ds(i * 16, 16)]
            return 0
        lax.fori_loop(0, hs // 16, add_body, 0)
        return 0
    lax.fori_loop(1, _NS, red_body, 0)

    @pl.when(is_f2c)
    def _():
        pltpu.sync_copy(acc, degc_hbm.at[pl.ds(t * hs, hs)])

    @pl.when(jnp.logical_not(is_f2c))
    def _():
        pltpu.sync_copy(acc, degf_hbm.at[pl.ds(t * hs, hs)])


def _densify(src, dst, ew_f2c, ew_c2f):
    mesh = plsc.VectorSubcoreMesh(core_axis_name="c", subcore_axis_name="s")
    ews = jnp.concatenate([ew_f2c, ew_c2f])
    zeros = jnp.zeros((_ZB,), jnp.float32)
    kern = pl.kernel(
        _densify_body,
        out_type=[
            jax.ShapeDtypeStruct((_MPAD,), jnp.float32),
            jax.ShapeDtypeStruct((_MPAD,), jnp.float32),
            jax.ShapeDtypeStruct((_HB,), jnp.float32),
            jax.ShapeDtypeStruct((_HB,), jnp.float32),
        ],
        mesh=mesh,
        compiler_params=pltpu.CompilerParams(needs_layout_passes=False),
        scratch_types=[
            pltpu.VMEM((_SROUND,), jnp.int32),
            pltpu.VMEM((_SROUND,), jnp.int32),
            pltpu.VMEM((_SROUND,), jnp.float32),
            pltpu.VMEM((_EBUF + 16,), jnp.int32),
            pltpu.VMEM((_EBUF + 16,), jnp.float32),
            pltpu.VMEM((_HB,), jnp.float32),
            pltpu.VMEM((_ZB,), jnp.float32),
            pltpu.VMEM((32,), jnp.int32),
            pltpu.VMEM((32,), jnp.int32),
            pltpu.VMEM((32,), jnp.int32),
            pltpu.VMEM((128,), jnp.float32),
            pltpu.VMEM((16,), jnp.float32),
            pltpu.VMEM((_HB // _NS,), jnp.float32),
            pltpu.VMEM((_HB // _NS,), jnp.float32),
            pltpu.VMEM_SHARED((_CH + 128,), jnp.float32),
            pltpu.VMEM_SHARED((_HB * _NS,), jnp.float32),
            pltpu.SemaphoreType.DMA,
        ],
    )
    af, ac, degc, degf = kern(src, dst, ews, zeros)
    a_f2c = af[:N_CELL * N_FEAT].reshape(N_CELL, N_FEAT)
    a_c2f = ac[:N_FEAT * N_CELL].reshape(N_FEAT, N_CELL)
    return a_f2c, a_c2f, degc[:N_CELL], degf[:N_FEAT]


def kernel(feat_ids, cell_ids, bf, src_f2c, dst_f2c, ew_f2c, ew_c2f, params):
    p = params
    a_f2c, a_c2f, deg_cell, deg_feat = _densify(src_f2c, dst_f2c, ew_f2c,
                                                ew_c2f)
    inv_deg_cell = (1.0 / jnp.maximum(deg_cell, 1.0))[:, None]
    inv_deg_feat = (1.0 / jnp.maximum(deg_feat, 1.0))[:, None]

    bn = bf.shape[1]
    bf_pad = jnp.pad(bf, ((0, 0), (0, 128 - bn)))
    we_pad = jnp.pad(p['W_extra'], ((0, 128 - bn), (0, 0)))

    hf = _init_feat(p['embed_feat'], p['in_W1'], p['in_b1'], p['in_g1'],
                    p['in_beta1'])
    hc = _init_cell(p['embed_cell'], bf_pad, we_pad, p['b_extra'], p['in_W0'],
                    p['in_b0'], p['in_g0'], p['in_beta0'])

    hist = []
    for layer in range(2):
        c = p['conv'][layer]
        new_hc = _conv(a_f2c, hf, hc, inv_deg_cell, c['f2c_Ws'], c['f2c_Wn'],
                       c['f2c_b'], c['norm_cell_g'], c['norm_cell_b'], bm=800)
        new_hf = _conv(a_c2f, hc, hf, inv_deg_feat, c['c2f_Ws'], c['c2f_Wn'],
                       c['c2f_b'], c['norm_feat_g'], c['norm_feat_b'], bm=200)
        hf, hc = new_hf, new_hc
        hist.append(hc)

    return _readout(hist[0], hist[1], p['ro_W0'], p['ro_b0'], p['ro_W1'],
                    p['ro_b1'], p['calib_a'], p['calib_b'])


# speedup vs baseline: 1.1632x; 1.1632x over previous
import functools

import jax
import jax.numpy as jnp
from jax import lax
from jax.experimental import pallas as pl
from jax.experimental.pallas import tpu as pltpu, tpu_sc as plsc

N_FEAT = 2000
N_CELL = 8000
E = 320000
H = 128
OUT = 128

_NS = 16
_EPT0 = 19968
_EPT15 = E - 15 * _EPT0
_EBUF = _EPT15
_PF = 2048
_PC = 8192
_MPAD = N_CELL * _PF
_CH = 819200
_NPASS = _MPAD // _CH
_WPT = _CH // _NS
_SROUND = 4992
_ZB = 2048
_HB = 8192
_DUMP = _CH


def _leaky(x):
    return jnp.where(x >= 0, x, 0.01 * x)


def _ln(x, g, b, eps=1e-5):
    mu = jnp.mean(x, axis=-1, keepdims=True)
    var = jnp.mean((x - mu) ** 2, axis=-1, keepdims=True)
    return (x - mu) / jnp.sqrt(var + eps) * g + b


def _init_feat_body(ef_ref, w_ref, b_ref, g_ref, beta_ref, out_ref):
    x = _leaky(ef_ref[...])
    x = jnp.maximum(jnp.dot(x, w_ref[...], preferred_element_type=jnp.float32)
                    + b_ref[...], 0.0)
    out_ref[...] = _ln(x, g_ref[...], beta_ref[...])


def _init_cell_body(ec_ref, bf_ref, we_ref, be_ref, w_ref, b_ref, g_ref,
                    beta_ref, out_ref):
    extra = _leaky(jnp.dot(bf_ref[...], we_ref[...],
                           preferred_element_type=jnp.float32) + be_ref[...])
    x = _leaky(ec_ref[...]) + extra
    x = jnp.maximum(jnp.dot(x, w_ref[...], preferred_element_type=jnp.float32)
                    + b_ref[...], 0.0)
    out_ref[...] = _ln(x, g_ref[...], beta_ref[...])


def _conv_body(a_ref, hsrc_ref, hdst_ref, deg_ref, ws_ref, wn_ref, b_ref,
               g_ref, beta_ref, out_ref):
    s = jnp.dot(a_ref[...], hsrc_ref[...], preferred_element_type=jnp.float32)
    h_neigh = s * (1.0 / jnp.maximum(deg_ref[...], 1.0))
    new = (jnp.dot(hdst_ref[...], ws_ref[...], preferred_element_type=jnp.float32)
           + jnp.dot(h_neigh, wn_ref[...], preferred_element_type=jnp.float32)
           + b_ref[...])
    out_ref[...] = jnp.maximum(_ln(new, g_ref[...], beta_ref[...]), 0.0)


def _readout_body(h1_ref, h2_ref, w0_ref, b0_ref, w1_ref, b1_ref, ca_ref,
                  cb_ref, out_ref):
    h = jnp.concatenate([h1_ref[...], h2_ref[...]], axis=1)
    h = jnp.maximum(jnp.dot(h, w0_ref[...], preferred_element_type=jnp.float32)
                    + b0_ref[...], 0.0)
    o = jnp.dot(h, w1_ref[...], preferred_element_type=jnp.float32) + b1_ref[...]
    out_ref[...] = o * ca_ref[...] + cb_ref[...]


def _full(shape):
    return pl.BlockSpec(shape, lambda *_: tuple(0 for _ in shape))


def _init_feat(ef, w, b, g, beta):
    return pl.pallas_call(
        _init_feat_body,
        out_shape=jax.ShapeDtypeStruct((N_FEAT, H), jnp.float32),
    )(ef, w, b[None, :], g[None, :], beta[None, :])


def _init_cell(ec, bf, we, be, w, b, g, beta):
    return pl.pallas_call(
        _init_cell_body,
        out_shape=jax.ShapeDtypeStruct((N_CELL, H), jnp.float32),
    )(ec, bf, we, be[None, :], w, b[None, :], g[None, :], beta[None, :])


def _conv(a, h_src, h_dst, deg, ws, wn, b, g, beta, bm):
    n_dst, n_src = a.shape
    grid = (n_dst // bm,)
    return pl.pallas_call(
        _conv_body,
        grid=grid,
        in_specs=[
            pl.BlockSpec((bm, n_src), lambda i: (i, 0)),
            pl.BlockSpec((n_src, H), lambda i: (0, 0)),
            pl.BlockSpec((bm, H), lambda i: (i, 0)),
            pl.BlockSpec((bm, 1), lambda i: (i, 0)),
            pl.BlockSpec((H, H), lambda i: (0, 0)),
            pl.BlockSpec((H, H), lambda i: (0, 0)),
            pl.BlockSpec((1, H), lambda i: (0, 0)),
            pl.BlockSpec((1, H), lambda i: (0, 0)),
            pl.BlockSpec((1, H), lambda i: (0, 0)),
        ],
        out_specs=pl.BlockSpec((bm, H), lambda i: (i, 0)),
        out_shape=jax.ShapeDtypeStruct((n_dst, H), jnp.float32),
    )(a, h_src, h_dst, deg[:, None], ws, wn, b[None, :], g[None, :], beta[None, :])


def _readout(h1, h2, w0, b0, w1, b1, ca, cb):
    return pl.pallas_call(
        _readout_body,
        out_shape=jax.ShapeDtypeStruct((N_CELL, OUT), jnp.float32),
    )(h1, h2, w0, b0[None, :], w1, b1[None, :], ca[None, :], cb[None, :])


def _densify_body(src_hbm, dst_hbm, ews_hbm, zeros_hbm,
                  af_hbm, ac_hbm, degc_hbm, degf_hbm,
                  src_v, dst_v, ew_v, binidx, binval, hist, zb,
                  cnt16, start16, cursor, stg, drain, acc, tmp,
                  chunk, hstage, sem):
    c = lax.axis_index("c")
    t = lax.axis_index("s")
    lane = lax.iota(jnp.int32, 16)
    is_f2c = c == 0
    stride = jnp.where(is_f2c, _PF, _PC)
    last = t == _NS - 1
    base_e = t * _EPT0

    zi = jnp.broadcast_to(jnp.int32(0), (16,))
    zf = jnp.broadcast_to(jnp.float32(0.0), (16,))
    ones_i = jnp.broadcast_to(jnp.int32(1), (16,))
    onef = jnp.broadcast_to(jnp.float32(1.0), (16,))
    cnt16[pl.ds(0, 16)] = zi
    cnt16[pl.ds(16, 16)] = zi

    def zset(i, _):
        zb[pl.ds(i * 16, 16)] = zf
        return 0
    lax.fori_loop(0, _ZB // 16, zset, 0)

    def hz(i, _):
        hist[pl.ds(i * 16, 16)] = zf
        return 0
    lax.fori_loop(0, _HB // 16, hz, 0)

    def edge_vec(i):
        o = i * 16
        s16 = src_v[pl.ds(o, 16)]
        d16 = dst_v[pl.ds(o, 16)]
        row = jnp.where(is_f2c, d16, s16)
        flat = row * stride + jnp.where(is_f2c, s16, d16)
        return row, flat

    def count_step(i, _):
        row, flat = edge_vec(i)
        bucket = flat // _CH
        plsc.addupdate_scatter(cnt16, [bucket], ones_i)
        plsc.addupdate_scatter(hist, [row], onef)
        return 0

    def place_step(i, _):
        row, flat = edge_vec(i)
        w16 = ew_v[pl.ds(i * 16, 16)]
        bucket = flat // _CH
        rel = flat - bucket * _CH
        rank, _l = plsc.scan_count(bucket)
        base = plsc.load_gather(cursor, [bucket])
        dest = base + rank - 1
        plsc.store_scatter(binidx, [dest], rel)
        plsc.store_scatter(binval, [dest], w16)
        plsc.addupdate_scatter(cursor, [bucket], ones_i)
        return 0

    def scan_edges(step, with_ew):
        def round_body(r, _):
            o = base_e + r * _SROUND
            pltpu.sync_copy(src_hbm.at[pl.ds(o, _SROUND)],
                            src_v.at[pl.ds(0, _SROUND)])
            pltpu.sync_copy(dst_hbm.at[pl.ds(o, _SROUND)],
                            dst_v.at[pl.ds(0, _SROUND)])
            if with_ew:
                pltpu.sync_copy(ews_hbm.at[pl.ds(c * E + o, _SROUND)],
                                ew_v.at[pl.ds(0, _SROUND)])
            lax.fori_loop(0, _SROUND // 16, step, 0)
            return 0
        lax.fori_loop(0, _EPT0 // _SROUND, round_body, 0)

        @pl.when(last)
        def _():
            o = base_e + (_EPT0 // _SROUND) * _SROUND
            pltpu.sync_copy(src_hbm.at[pl.ds(o, 512)],
                            src_v.at[pl.ds(0, 512)])
            pltpu.sync_copy(dst_hbm.at[pl.ds(o, 512)],
                            dst_v.at[pl.ds(0, 512)])
            if with_ew:
                pltpu.sync_copy(ews_hbm.at[pl.ds(c * E + o, 512)],
                                ew_v.at[pl.ds(0, 512)])
            lax.fori_loop(0, 512 // 16, step, 0)

    scan_edges(count_step, with_ew=False)

    c0 = cnt16[pl.ds(0, 16)]
    c1 = cnt16[pl.ds(16, 16)]
    st0 = plsc.cumsum(c0) - c0
    st1 = plsc.cumsum(c1) - c1 + jnp.sum(c0)
    start16[pl.ds(0, 16)] = st0
    start16[pl.ds(16, 16)] = st1
    cursor[pl.ds(0, 16)] = st0
    cursor[pl.ds(16, 16)] = st1

    scan_edges(place_step, with_ew=True)

    def pass_body(p, _):
        def zcp(j, _):
            pltpu.async_copy(zb, chunk.at[pl.ds(t * _WPT + j * _ZB, _ZB)],
                             sem)
            return 0
        lax.fori_loop(0, _WPT // _ZB, zcp, 0)

        @pl.when(last)
        def _():
            pltpu.async_copy(zb.at[pl.ds(0, 128)],
                             chunk.at[pl.ds(_CH, 128)], sem)

        def zdrain(j, _):
            pltpu.make_async_copy(zeros_hbm, zb, sem).wait()
            return 0
        lax.fori_loop(0, _WPT // _ZB, zdrain, 0)

        @pl.when(last)
        def _():
            pltpu.make_async_copy(zeros_hbm.at[pl.ds(0, 128)],
                                  zb.at[pl.ds(0, 128)], sem).wait()
        plsc.subcore_barrier()

        sv0 = start16[pl.ds(0, 16)]
        sv1 = start16[pl.ds(16, 16)]
        cv0 = cnt16[pl.ds(0, 16)]
        cv1 = cnt16[pl.ds(16, 16)]
        start_p = jnp.sum(jnp.where(lane == p, sv0, 0)
                          + jnp.where(lane == p - 16, sv1, 0))
        cnt_p = jnp.sum(jnp.where(lane == p, cv0, 0)
                        + jnp.where(lane == p - 16, cv1, 0))
        nsteps = (cnt_p + 15) >> 4

        def sc_body(k, _):
            @pl.when(k >= 8)
            def _():
                pltpu.make_async_copy(zeros_hbm.at[pl.ds(0, 16)], drain,
                                      sem).wait()
            off = start_p + k * 16
            idx16 = binidx[pl.ds(off, 16)]
            val16 = binval[pl.ds(off, 16)]
            m = (k * 16 + lane) < cnt_p
            idx16 = jnp.where(m, idx16, _DUMP)
            val16 = jnp.where(m, val16, 0.0)
            so = (k & 7) * 16
            stg[pl.ds(so, 16)] = val16
            pltpu.async_copy(stg.at[pl.ds(so, 16)], chunk.at[idx16], sem,
                             add=True)
            return 0
        lax.fori_loop(0, nsteps, sc_body, 0)

        def drain_body(j, _):
            pltpu.make_async_copy(zeros_hbm.at[pl.ds(0, 16)], drain,
                                  sem).wait()
            return 0
        lax.fori_loop(0, jnp.minimum(nsteps, 8), drain_body, 0)

        plsc.subcore_barrier()
        hbm_off = p * _CH + t * _WPT

        @pl.when(is_f2c)
        def _():
            pltpu.sync_copy(chunk.at[pl.ds(t * _WPT, _WPT)],
                            af_hbm.at[pl.ds(hbm_off, _WPT)])

        @pl.when(jnp.logical_not(is_f2c))
        def _():
            pltpu.sync_copy(chunk.at[pl.ds(t * _WPT, _WPT)],
                            ac_hbm.at[pl.ds(hbm_off, _WPT)])
        plsc.subcore_barrier()
        return 0
    lax.fori_loop(0, _NPASS, pass_body, 0)

    hs = _HB // _NS
    pltpu.sync_copy(hist, hstage.at[pl.ds(t * _HB, _HB)])
    plsc.subcore_barrier()
    pltpu.sync_copy(hstage.at[pl.ds(t * hs, hs)], acc)

    def red_body(j, _):
        pltpu.sync_copy(hstage.at[pl.ds(j * _HB + t * hs, hs)], tmp)

        def add_body(i, _):
            acc[pl.ds(i * 16, 16)] = acc[pl.ds(i * 16, 16)] + tmp[pl.ds(i * 16, 16)]
            return 0
        lax.fori_loop(0, hs // 16, add_body, 0)
        return 0
    lax.fori_loop(1, _NS, red_body, 0)

    @pl.when(is_f2c)
    def _():
        pltpu.sync_copy(acc, degc_hbm.at[pl.ds(t * hs, hs)])

    @pl.when(jnp.logical_not(is_f2c))
    def _():
        pltpu.sync_copy(acc, degf_hbm.at[pl.ds(t * hs, hs)])


def _densify(src, dst, ew_f2c, ew_c2f):
    mesh = plsc.VectorSubcoreMesh(core_axis_name="c", subcore_axis_name="s")
    ews = jnp.concatenate([ew_f2c, ew_c2f])
    zeros = jnp.zeros((_ZB,), jnp.float32)
    kern = pl.kernel(
        _densify_body,
        out_type=[
            jax.ShapeDtypeStruct((_MPAD,), jnp.float32),
            jax.ShapeDtypeStruct((_MPAD,), jnp.float32),
            jax.ShapeDtypeStruct((_HB,), jnp.float32),
            jax.ShapeDtypeStruct((_HB,), jnp.float32),
        ],
        mesh=mesh,
        compiler_params=pltpu.CompilerParams(needs_layout_passes=False),
        scratch_types=[
            pltpu.VMEM((_SROUND,), jnp.int32),
            pltpu.VMEM((_SROUND,), jnp.int32),
            pltpu.VMEM((_SROUND,), jnp.float32),
            pltpu.VMEM((_EBUF + 16,), jnp.int32),
            pltpu.VMEM((_EBUF + 16,), jnp.float32),
            pltpu.VMEM((_HB,), jnp.float32),
            pltpu.VMEM((_ZB,), jnp.float32),
            pltpu.VMEM((32,), jnp.int32),
            pltpu.VMEM((32,), jnp.int32),
            pltpu.VMEM((32,), jnp.int32),
            pltpu.VMEM((128,), jnp.float32),
            pltpu.VMEM((16,), jnp.float32),
            pltpu.VMEM((_HB // _NS,), jnp.float32),
            pltpu.VMEM((_HB // _NS,), jnp.float32),
            pltpu.VMEM_SHARED((_CH + 128,), jnp.float32),
            pltpu.VMEM_SHARED((_HB * _NS,), jnp.float32),
            pltpu.SemaphoreType.DMA,
        ],
    )
    af, ac, degc, degf = kern(src, dst, ews, zeros)
    a_f2c = af.reshape(N_CELL, _PF)
    a_c2f = ac.reshape(N_FEAT, _PC)
    return a_f2c, a_c2f, degc[:N_CELL], degf[:N_FEAT]


def kernel(feat_ids, cell_ids, bf, src_f2c, dst_f2c, ew_f2c, ew_c2f, params):
    p = params
    a_f2c, a_c2f, deg_cell, deg_feat = _densify(src_f2c, dst_f2c, ew_f2c,
                                                ew_c2f)

    bn = bf.shape[1]
    bf_pad = jnp.pad(bf, ((0, 0), (0, 128 - bn)))
    we_pad = jnp.pad(p['W_extra'], ((0, 128 - bn), (0, 0)))

    hf = _init_feat(p['embed_feat'], p['in_W1'], p['in_b1'], p['in_g1'],
                    p['in_beta1'])
    hc = _init_cell(p['embed_cell'], bf_pad, we_pad, p['b_extra'], p['in_W0'],
                    p['in_b0'], p['in_g0'], p['in_beta0'])

    hist = []
    for layer in range(2):
        c = p['conv'][layer]
        hf_p = jnp.pad(hf, ((0, _PF - N_FEAT), (0, 0)))
        hc_p = jnp.pad(hc, ((0, _PC - N_CELL), (0, 0)))
        new_hc = _conv(a_f2c, hf_p, hc, deg_cell, c['f2c_Ws'], c['f2c_Wn'],
                       c['f2c_b'], c['norm_cell_g'], c['norm_cell_b'], bm=800)
        new_hf = _conv(a_c2f, hc_p, hf, deg_feat, c['c2f_Ws'], c['c2f_Wn'],
                       c['c2f_b'], c['norm_feat_g'], c['norm_feat_b'], bm=200)
        hf, hc = new_hf, new_hc
        hist.append(hc)

    return _readout(hist[0], hist[1], p['ro_W0'], p['ro_b0'], p['ro_W1'],
                    p['ro_b1'], p['calib_a'], p['calib_b'])


# speedup vs baseline: 1.4779x; 1.2706x over previous
import functools

import jax
import jax.numpy as jnp
from jax import lax
from jax.experimental import pallas as pl
from jax.experimental.pallas import tpu as pltpu, tpu_sc as plsc

N_FEAT = 2000
N_CELL = 8000
E = 320000
H = 128
OUT = 128

_NS = 16
_EPT0 = 19968
_EPT15 = E - 15 * _EPT0
_EBUF = _EPT15
_PF = 2048
_PC = 8192
_MPAD = N_CELL * _PF
_CH = 819200
_NPASS = _MPAD // _CH
_WPT = _CH // _NS
_SROUND = 4992
_ZB = 2048
_HB = 8192
_DUMP = _CH


def _leaky(x):
    return jnp.where(x >= 0, x, 0.01 * x)


def _ln(x, g, b, eps=1e-5):
    mu = jnp.mean(x, axis=-1, keepdims=True)
    var = jnp.mean((x - mu) ** 2, axis=-1, keepdims=True)
    return (x - mu) / jnp.sqrt(var + eps) * g + b


def _init_feat_body(ef_ref, w_ref, b_ref, g_ref, beta_ref, out_ref):
    x = _leaky(ef_ref[...])
    x = jnp.maximum(jnp.dot(x, w_ref[...], preferred_element_type=jnp.float32)
                    + b_ref[...], 0.0)
    out_ref[...] = _ln(x, g_ref[...], beta_ref[...])


def _init_cell_body(ec_ref, bf_ref, we_ref, be_ref, w_ref, b_ref, g_ref,
                    beta_ref, out_ref):
    extra = _leaky(jnp.dot(bf_ref[...], we_ref[...],
                           preferred_element_type=jnp.float32) + be_ref[...])
    x = _leaky(ec_ref[...]) + extra
    x = jnp.maximum(jnp.dot(x, w_ref[...], preferred_element_type=jnp.float32)
                    + b_ref[...], 0.0)
    out_ref[...] = _ln(x, g_ref[...], beta_ref[...])


def _conv_body(a_ref, hsrc_ref, hdst_ref, deg_ref, ws_ref, wn_ref, b_ref,
               g_ref, beta_ref, out_ref):
    n_src = hsrc_ref.shape[0]
    bm = hdst_ref.shape[0]
    a = a_ref[...].reshape(bm, n_src)
    s = jnp.dot(a, hsrc_ref[...], preferred_element_type=jnp.float32)
    h_neigh = s * (1.0 / jnp.maximum(deg_ref[...], 1.0))
    new = (jnp.dot(hdst_ref[...], ws_ref[...], preferred_element_type=jnp.float32)
           + jnp.dot(h_neigh, wn_ref[...], preferred_element_type=jnp.float32)
           + b_ref[...])
    out_ref[...] = jnp.maximum(_ln(new, g_ref[...], beta_ref[...]), 0.0)


def _readout_body(h1_ref, h2_ref, w0_ref, b0_ref, w1_ref, b1_ref, ca_ref,
                  cb_ref, out_ref):
    h = jnp.concatenate([h1_ref[...], h2_ref[...]], axis=1)
    h = jnp.maximum(jnp.dot(h, w0_ref[...], preferred_element_type=jnp.float32)
                    + b0_ref[...], 0.0)
    o = jnp.dot(h, w1_ref[...], preferred_element_type=jnp.float32) + b1_ref[...]
    out_ref[...] = o * ca_ref[...] + cb_ref[...]


def _full(shape):
    return pl.BlockSpec(shape, lambda *_: tuple(0 for _ in shape))


def _init_feat(ef, w, b, g, beta):
    return pl.pallas_call(
        _init_feat_body,
        out_shape=jax.ShapeDtypeStruct((N_FEAT, H), jnp.float32),
    )(ef, w, b[None, :], g[None, :], beta[None, :])


def _init_cell(ec, bf, we, be, w, b, g, beta):
    return pl.pallas_call(
        _init_cell_body,
        out_shape=jax.ShapeDtypeStruct((N_CELL, H), jnp.float32),
    )(ec, bf, we, be[None, :], w, b[None, :], g[None, :], beta[None, :])


def _conv(a, h_src, h_dst, deg, ws, wn, b, g, beta, bm, n_dst, n_src):
    grid = (n_dst // bm,)
    rows_blk = bm * n_src // 128
    return pl.pallas_call(
        _conv_body,
        grid=grid,
        in_specs=[
            pl.BlockSpec((rows_blk, 128), lambda i: (i, 0)),
            pl.BlockSpec((n_src, H), lambda i: (0, 0)),
            pl.BlockSpec((bm, H), lambda i: (i, 0)),
            pl.BlockSpec((bm, 1), lambda i: (i, 0)),
            pl.BlockSpec((H, H), lambda i: (0, 0)),
            pl.BlockSpec((H, H), lambda i: (0, 0)),
            pl.BlockSpec((1, H), lambda i: (0, 0)),
            pl.BlockSpec((1, H), lambda i: (0, 0)),
            pl.BlockSpec((1, H), lambda i: (0, 0)),
        ],
        out_specs=pl.BlockSpec((bm, H), lambda i: (i, 0)),
        out_shape=jax.ShapeDtypeStruct((n_dst, H), jnp.float32),
    )(a, h_src, h_dst, deg[:, None], ws, wn, b[None, :], g[None, :], beta[None, :])


def _readout(h1, h2, w0, b0, w1, b1, ca, cb):
    return pl.pallas_call(
        _readout_body,
        out_shape=jax.ShapeDtypeStruct((N_CELL, OUT), jnp.float32),
    )(h1, h2, w0, b0[None, :], w1, b1[None, :], ca[None, :], cb[None, :])


def _densify_body(src_hbm, dst_hbm, ews_hbm, zeros_hbm,
                  af_hbm, ac_hbm, degc_hbm, degf_hbm,
                  src_v, dst_v, ew_v, binidx, binval, hist, zb,
                  cnt16, start16, cursor, stg, drain, acc, tmp,
                  chunk, hstage, sem):
    c = lax.axis_index("c")
    t = lax.axis_index("s")
    lane = lax.iota(jnp.int32, 16)
    is_f2c = c == 0
    stride = jnp.where(is_f2c, _PF, _PC)
    last = t == _NS - 1
    base_e = t * _EPT0

    zi = jnp.broadcast_to(jnp.int32(0), (16,))
    zf = jnp.broadcast_to(jnp.float32(0.0), (16,))
    ones_i = jnp.broadcast_to(jnp.int32(1), (16,))
    onef = jnp.broadcast_to(jnp.float32(1.0), (16,))
    cnt16[pl.ds(0, 16)] = zi
    cnt16[pl.ds(16, 16)] = zi

    def zset(i, _):
        zb[pl.ds(i * 16, 16)] = zf
        return 0
    lax.fori_loop(0, _ZB // 16, zset, 0)

    def hz(i, _):
        hist[pl.ds(i * 16, 16)] = zf
        return 0
    lax.fori_loop(0, _HB // 16, hz, 0)

    def edge_vec(i):
        o = i * 16
        s16 = src_v[pl.ds(o, 16)]
        d16 = dst_v[pl.ds(o, 16)]
        row = jnp.where(is_f2c, d16, s16)
        flat = row * stride + jnp.where(is_f2c, s16, d16)
        return row, flat

    def count_step(i, _):
        row, flat = edge_vec(i)
        bucket = flat // _CH
        plsc.addupdate_scatter(cnt16, [bucket], ones_i)
        plsc.addupdate_scatter(hist, [row], onef)
        return 0

    def place_step(i, _):
        row, flat = edge_vec(i)
        w16 = ew_v[pl.ds(i * 16, 16)]
        bucket = flat // _CH
        rel = flat - bucket * _CH
        rank, _l = plsc.scan_count(bucket)
        base = plsc.load_gather(cursor, [bucket])
        dest = base + rank - 1
        plsc.store_scatter(binidx, [dest], rel)
        plsc.store_scatter(binval, [dest], w16)
        plsc.addupdate_scatter(cursor, [bucket], ones_i)
        return 0

    def scan_edges(step, with_ew):
        def round_body(r, _):
            o = base_e + r * _SROUND
            pltpu.sync_copy(src_hbm.at[pl.ds(o, _SROUND)],
                            src_v.at[pl.ds(0, _SROUND)])
            pltpu.sync_copy(dst_hbm.at[pl.ds(o, _SROUND)],
                            dst_v.at[pl.ds(0, _SROUND)])
            if with_ew:
                pltpu.sync_copy(ews_hbm.at[pl.ds(c * E + o, _SROUND)],
                                ew_v.at[pl.ds(0, _SROUND)])
            lax.fori_loop(0, _SROUND // 16, step, 0)
            return 0
        lax.fori_loop(0, _EPT0 // _SROUND, round_body, 0)

        @pl.when(last)
        def _():
            o = base_e + (_EPT0 // _SROUND) * _SROUND
            pltpu.sync_copy(src_hbm.at[pl.ds(o, 512)],
                            src_v.at[pl.ds(0, 512)])
            pltpu.sync_copy(dst_hbm.at[pl.ds(o, 512)],
                            dst_v.at[pl.ds(0, 512)])
            if with_ew:
                pltpu.sync_copy(ews_hbm.at[pl.ds(c * E + o, 512)],
                                ew_v.at[pl.ds(0, 512)])
            lax.fori_loop(0, 512 // 16, step, 0)

    scan_edges(count_step, with_ew=False)

    c0 = cnt16[pl.ds(0, 16)]
    c1 = cnt16[pl.ds(16, 16)]
    st0 = plsc.cumsum(c0) - c0
    st1 = plsc.cumsum(c1) - c1 + jnp.sum(c0)
    start16[pl.ds(0, 16)] = st0
    start16[pl.ds(16, 16)] = st1
    cursor[pl.ds(0, 16)] = st0
    cursor[pl.ds(16, 16)] = st1

    scan_edges(place_step, with_ew=True)

    def pass_body(p, _):
        def zcp(j, _):
            pltpu.async_copy(zb, chunk.at[pl.ds(t * _WPT + j * _ZB, _ZB)],
                             sem)
            return 0
        lax.fori_loop(0, _WPT // _ZB, zcp, 0)

        @pl.when(last)
        def _():
            pltpu.async_copy(zb.at[pl.ds(0, 128)],
                             chunk.at[pl.ds(_CH, 128)], sem)

        def zdrain(j, _):
            pltpu.make_async_copy(zeros_hbm, zb, sem).wait()
            return 0
        lax.fori_loop(0, _WPT // _ZB, zdrain, 0)

        @pl.when(last)
        def _():
            pltpu.make_async_copy(zeros_hbm.at[pl.ds(0, 128)],
                                  zb.at[pl.ds(0, 128)], sem).wait()
        plsc.subcore_barrier()

        sv0 = start16[pl.ds(0, 16)]
        sv1 = start16[pl.ds(16, 16)]
        cv0 = cnt16[pl.ds(0, 16)]
        cv1 = cnt16[pl.ds(16, 16)]
        start_p = jnp.sum(jnp.where(lane == p, sv0, 0)
                          + jnp.where(lane == p - 16, sv1, 0))
        cnt_p = jnp.sum(jnp.where(lane == p, cv0, 0)
                        + jnp.where(lane == p - 16, cv1, 0))
        nsteps = (cnt_p + 15) >> 4

        def sc_body(k, _):
            @pl.when(k >= 8)
            def _():
                pltpu.make_async_copy(zeros_hbm.at[pl.ds(0, 16)], drain,
                                      sem).wait()
            off = start_p + k * 16
            idx16 = binidx[pl.ds(off, 16)]
            val16 = binval[pl.ds(off, 16)]
            m = (k * 16 + lane) < cnt_p
            idx16 = jnp.where(m, idx16, _DUMP)
            val16 = jnp.where(m, val16, 0.0)
            so = (k & 7) * 16
            stg[pl.ds(so, 16)] = val16
            pltpu.async_copy(stg.at[pl.ds(so, 16)], chunk.at[idx16], sem,
                             add=True)
            return 0
        lax.fori_loop(0, nsteps, sc_body, 0)

        def drain_body(j, _):
            pltpu.make_async_copy(zeros_hbm.at[pl.ds(0, 16)], drain,
                                  sem).wait()
            return 0
        lax.fori_loop(0, jnp.minimum(nsteps, 8), drain_body, 0)

        plsc.subcore_barrier()
        hbm_off = p * _CH + t * _WPT

        @pl.when(is_f2c)
        def _():
            pltpu.sync_copy(chunk.at[pl.ds(t * _WPT, _WPT)],
                            af_hbm.at[pl.ds(hbm_off, _WPT)])

        @pl.when(jnp.logical_not(is_f2c))
        def _():
            pltpu.sync_copy(chunk.at[pl.ds(t * _WPT, _WPT)],
                            ac_hbm.at[pl.ds(hbm_off, _WPT)])
        plsc.subcore_barrier()
        return 0
    lax.fori_loop(0, _NPASS, pass_body, 0)

    hs = _HB // _NS
    pltpu.sync_copy(hist, hstage.at[pl.ds(t * _HB, _HB)])
    plsc.subcore_barrier()
    pltpu.sync_copy(hstage.at[pl.ds(t * hs, hs)], acc)

    def red_body(j, _):
        pltpu.sync_copy(hstage.at[pl.ds(j * _HB + t * hs, hs)], tmp)

        def add_body(i, _):
            acc[pl.ds(i * 16, 16)] = acc[pl.ds(i * 16, 16)] + tmp[pl.ds(i * 16, 16)]
            return 0
        lax.fori_loop(0, hs // 16, add_body, 0)
        return 0
    lax.fori_loop(1, _NS, red_body, 0)

    @pl.when(is_f2c)
    def _():
        pltpu.sync_copy(acc, degc_hbm.at[pl.ds(t * hs, hs)])

    @pl.when(jnp.logical_not(is_f2c))
    def _():
        pltpu.sync_copy(acc, degf_hbm.at[pl.ds(t * hs, hs)])


def _densify(src, dst, ew_f2c, ew_c2f):
    mesh = plsc.VectorSubcoreMesh(core_axis_name="c", subcore_axis_name="s")
    ews = jnp.concatenate([ew_f2c, ew_c2f])
    zeros = jnp.zeros((_ZB,), jnp.float32)
    kern = pl.kernel(
        _densify_body,
        out_type=[
            jax.ShapeDtypeStruct((_MPAD,), jnp.float32),
            jax.ShapeDtypeStruct((_MPAD,), jnp.float32),
            jax.ShapeDtypeStruct((_HB,), jnp.float32),
            jax.ShapeDtypeStruct((_HB,), jnp.float32),
        ],
        mesh=mesh,
        compiler_params=pltpu.CompilerParams(needs_layout_passes=False),
        scratch_types=[
            pltpu.VMEM((_SROUND,), jnp.int32),
            pltpu.VMEM((_SROUND,), jnp.int32),
            pltpu.VMEM((_SROUND,), jnp.float32),
            pltpu.VMEM((_EBUF + 16,), jnp.int32),
            pltpu.VMEM((_EBUF + 16,), jnp.float32),
            pltpu.VMEM((_HB,), jnp.float32),
            pltpu.VMEM((_ZB,), jnp.float32),
            pltpu.VMEM((32,), jnp.int32),
            pltpu.VMEM((32,), jnp.int32),
            pltpu.VMEM((32,), jnp.int32),
            pltpu.VMEM((128,), jnp.float32),
            pltpu.VMEM((16,), jnp.float32),
            pltpu.VMEM((_HB // _NS,), jnp.float32),
            pltpu.VMEM((_HB // _NS,), jnp.float32),
            pltpu.VMEM_SHARED((_CH + 128,), jnp.float32),
            pltpu.VMEM_SHARED((_HB * _NS,), jnp.float32),
            pltpu.SemaphoreType.DMA,
        ],
    )
    af, ac, degc, degf = kern(src, dst, ews, zeros)
    a_f2c = af.reshape(_MPAD // 128, 128)
    a_c2f = ac.reshape(_MPAD // 128, 128)
    return a_f2c, a_c2f, degc[:N_CELL], degf[:N_FEAT]


def kernel(feat_ids, cell_ids, bf, src_f2c, dst_f2c, ew_f2c, ew_c2f, params):
    p = params
    a_f2c, a_c2f, deg_cell, deg_feat = _densify(src_f2c, dst_f2c, ew_f2c,
                                                ew_c2f)

    bn = bf.shape[1]
    bf_pad = jnp.pad(bf, ((0, 0), (0, 128 - bn)))
    we_pad = jnp.pad(p['W_extra'], ((0, 128 - bn), (0, 0)))

    hf = _init_feat(p['embed_feat'], p['in_W1'], p['in_b1'], p['in_g1'],
                    p['in_beta1'])
    hc = _init_cell(p['embed_cell'], bf_pad, we_pad, p['b_extra'], p['in_W0'],
                    p['in_b0'], p['in_g0'], p['in_beta0'])

    hist = []
    for layer in range(2):
        c = p['conv'][layer]
        hf_p = jnp.pad(hf, ((0, _PF - N_FEAT), (0, 0)))
        hc_p = jnp.pad(hc, ((0, _PC - N_CELL), (0, 0)))
        new_hc = _conv(a_f2c, hf_p, hc, deg_cell, c['f2c_Ws'], c['f2c_Wn'],
                       c['f2c_b'], c['norm_cell_g'], c['norm_cell_b'],
                       bm=800, n_dst=N_CELL, n_src=_PF)
        new_hf = _conv(a_c2f, hc_p, hf, deg_feat, c['c2f_Ws'], c['c2f_Wn'],
                       c['c2f_b'], c['norm_feat_g'], c['norm_feat_b'],
                       bm=200, n_dst=N_FEAT, n_src=_PC)
        hf, hc = new_hf, new_hc
        hist.append(hc)

    return _readout(hist[0], hist[1], p['ro_W0'], p['ro_b0'], p['ro_W1'],
                    p['ro_b1'], p['calib_a'], p['calib_b'])


# speedup vs baseline: 1.4924x; 1.0099x over previous
import functools

import jax
import jax.numpy as jnp
from jax import lax
from jax.experimental import pallas as pl
from jax.experimental.pallas import tpu as pltpu, tpu_sc as plsc

N_FEAT = 2000
N_CELL = 8000
E = 320000
H = 128
OUT = 128

_NS = 16
_EPT0 = 19968
_EPT15 = E - 15 * _EPT0
_EBUF = _EPT15
_PF = 2048
_PC = 8192
_MPAD = N_CELL * _PF
_CH = 819200
_NPASS = _MPAD // _CH
_WPT = _CH // _NS
_SROUND = 4992
_ZB = 2048
_HB = 8192
_DUMP = _CH


def _leaky(x):
    return jnp.where(x >= 0, x, 0.01 * x)


def _ln(x, g, b, eps=1e-5):
    mu = jnp.mean(x, axis=-1, keepdims=True)
    var = jnp.mean((x - mu) ** 2, axis=-1, keepdims=True)
    return (x - mu) / jnp.sqrt(var + eps) * g + b


def _init_feat_body(ef_ref, w_ref, b_ref, g_ref, beta_ref, out_ref):
    x = _leaky(ef_ref[...])
    x = jnp.maximum(jnp.dot(x, w_ref[...], preferred_element_type=jnp.float32)
                    + b_ref[...], 0.0)
    out_ref[...] = _ln(x, g_ref[...], beta_ref[...])


def _init_cell_body(ec_ref, bf_ref, we_ref, be_ref, w_ref, b_ref, g_ref,
                    beta_ref, out_ref):
    extra = _leaky(jnp.dot(bf_ref[...], we_ref[...],
                           preferred_element_type=jnp.float32) + be_ref[...])
    x = _leaky(ec_ref[...]) + extra
    x = jnp.maximum(jnp.dot(x, w_ref[...], preferred_element_type=jnp.float32)
                    + b_ref[...], 0.0)
    out_ref[...] = _ln(x, g_ref[...], beta_ref[...])


def _conv_body(a_ref, hsrc_ref, hdst_ref, deg_ref, ws_ref, wn_ref, b_ref,
               g_ref, beta_ref, out_ref):
    n_src = hsrc_ref.shape[0]
    bm = hdst_ref.shape[0]
    a = a_ref[...].reshape(bm, n_src)
    s = jnp.dot(a, hsrc_ref[...], preferred_element_type=jnp.float32)
    h_neigh = s * (1.0 / jnp.maximum(deg_ref[...], 1.0))
    new = (jnp.dot(hdst_ref[...], ws_ref[...], preferred_element_type=jnp.float32)
           + jnp.dot(h_neigh, wn_ref[...], preferred_element_type=jnp.float32)
           + b_ref[...])
    out_ref[...] = jnp.maximum(_ln(new, g_ref[...], beta_ref[...]), 0.0)


def _readout_body(h1_ref, h2_ref, w0_ref, b0_ref, w1_ref, b1_ref, ca_ref,
                  cb_ref, out_ref):
    h = jnp.concatenate([h1_ref[...], h2_ref[...]], axis=1)
    h = jnp.maximum(jnp.dot(h, w0_ref[...], preferred_element_type=jnp.float32)
                    + b0_ref[...], 0.0)
    o = jnp.dot(h, w1_ref[...], preferred_element_type=jnp.float32) + b1_ref[...]
    out_ref[...] = o * ca_ref[...] + cb_ref[...]


def _full(shape):
    return pl.BlockSpec(shape, lambda *_: tuple(0 for _ in shape))


def _init_feat(ef, w, b, g, beta):
    return pl.pallas_call(
        _init_feat_body,
        out_shape=jax.ShapeDtypeStruct((N_FEAT, H), jnp.float32),
    )(ef, w, b[None, :], g[None, :], beta[None, :])


def _init_cell(ec, bf, we, be, w, b, g, beta):
    return pl.pallas_call(
        _init_cell_body,
        out_shape=jax.ShapeDtypeStruct((N_CELL, H), jnp.float32),
    )(ec, bf, we, be[None, :], w, b[None, :], g[None, :], beta[None, :])


def _conv(a, h_src, h_dst, deg, ws, wn, b, g, beta, bm, n_dst, n_src):
    grid = (n_dst // bm,)
    rows_blk = bm * n_src // 128
    return pl.pallas_call(
        _conv_body,
        grid=grid,
        in_specs=[
            pl.BlockSpec((rows_blk, 128), lambda i: (i, 0)),
            pl.BlockSpec((n_src, H), lambda i: (0, 0)),
            pl.BlockSpec((bm, H), lambda i: (i, 0)),
            pl.BlockSpec((bm, 1), lambda i: (i, 0)),
            pl.BlockSpec((H, H), lambda i: (0, 0)),
            pl.BlockSpec((H, H), lambda i: (0, 0)),
            pl.BlockSpec((1, H), lambda i: (0, 0)),
            pl.BlockSpec((1, H), lambda i: (0, 0)),
            pl.BlockSpec((1, H), lambda i: (0, 0)),
        ],
        out_specs=pl.BlockSpec((bm, H), lambda i: (i, 0)),
        out_shape=jax.ShapeDtypeStruct((n_dst, H), jnp.float32),
    )(a, h_src, h_dst, deg[:, None], ws, wn, b[None, :], g[None, :], beta[None, :])


def _readout(h1, h2, w0, b0, w1, b1, ca, cb):
    return pl.pallas_call(
        _readout_body,
        out_shape=jax.ShapeDtypeStruct((N_CELL, OUT), jnp.float32),
    )(h1, h2, w0, b0[None, :], w1, b1[None, :], ca[None, :], cb[None, :])


def _densify_body(src_hbm, dst_hbm, ews_hbm, zeros_hbm,
                  af_hbm, ac_hbm, degc_hbm, degf_hbm,
                  src_v, dst_v, ew_v, binidx, binval, hist, zb,
                  cnt16, start16, cursor, stg, drain, acc, tmp,
                  chunk, hstage, sem):
    c = lax.axis_index("c")
    t = lax.axis_index("s")
    lane = lax.iota(jnp.int32, 16)
    is_f2c = c == 0
    stride = jnp.where(is_f2c, _PF, _PC)
    last = t == _NS - 1
    base_e = t * _EPT0

    zi = jnp.broadcast_to(jnp.int32(0), (16,))
    zf = jnp.broadcast_to(jnp.float32(0.0), (16,))
    ones_i = jnp.broadcast_to(jnp.int32(1), (16,))
    onef = jnp.broadcast_to(jnp.float32(1.0), (16,))
    cnt16[pl.ds(0, 16)] = zi
    cnt16[pl.ds(16, 16)] = zi

    def zset(i, _):
        zb[pl.ds(i * 16, 16)] = zf
        return 0
    lax.fori_loop(0, _ZB // 16, zset, 0)

    def hz(i, _):
        hist[pl.ds(i * 16, 16)] = zf
        return 0
    lax.fori_loop(0, _HB // 16, hz, 0)

    def edge_vec(i):
        o = i * 16
        s16 = src_v[pl.ds(o, 16)]
        d16 = dst_v[pl.ds(o, 16)]
        row = jnp.where(is_f2c, d16, s16)
        flat = row * stride + jnp.where(is_f2c, s16, d16)
        return row, flat

    def count_step(i, _):
        row, flat = edge_vec(i)
        bucket = flat // _CH
        plsc.addupdate_scatter(cnt16, [bucket], ones_i)
        plsc.addupdate_scatter(hist, [row], onef)
        return 0

    def place_step(i, _):
        row, flat = edge_vec(i)
        w16 = ew_v[pl.ds(i * 16, 16)]
        bucket = flat // _CH
        rel = flat - bucket * _CH
        rank, _l = plsc.scan_count(bucket)
        base = plsc.load_gather(cursor, [bucket])
        dest = base + rank - 1
        plsc.store_scatter(binidx, [dest], rel)
        plsc.store_scatter(binval, [dest], w16)
        plsc.addupdate_scatter(cursor, [bucket], ones_i)
        return 0

    def scan_edges(step, with_ew):
        def round_body(r, _):
            o = base_e + r * _SROUND
            pltpu.sync_copy(src_hbm.at[pl.ds(o, _SROUND)],
                            src_v.at[pl.ds(0, _SROUND)])
            pltpu.sync_copy(dst_hbm.at[pl.ds(o, _SROUND)],
                            dst_v.at[pl.ds(0, _SROUND)])
            if with_ew:
                pltpu.sync_copy(ews_hbm.at[pl.ds(c * E + o, _SROUND)],
                                ew_v.at[pl.ds(0, _SROUND)])
            lax.fori_loop(0, _SROUND // 16, step, 0)
            return 0
        lax.fori_loop(0, _EPT0 // _SROUND, round_body, 0)

        @pl.when(last)
        def _():
            o = base_e + (_EPT0 // _SROUND) * _SROUND
            pltpu.sync_copy(src_hbm.at[pl.ds(o, 512)],
                            src_v.at[pl.ds(0, 512)])
            pltpu.sync_copy(dst_hbm.at[pl.ds(o, 512)],
                            dst_v.at[pl.ds(0, 512)])
            if with_ew:
                pltpu.sync_copy(ews_hbm.at[pl.ds(c * E + o, 512)],
                                ew_v.at[pl.ds(0, 512)])
            lax.fori_loop(0, 512 // 16, step, 0)

    scan_edges(count_step, with_ew=False)

    c0 = cnt16[pl.ds(0, 16)]
    c1 = cnt16[pl.ds(16, 16)]
    st0 = plsc.cumsum(c0) - c0
    st1 = plsc.cumsum(c1) - c1 + jnp.sum(c0)
    start16[pl.ds(0, 16)] = st0
    start16[pl.ds(16, 16)] = st1
    cursor[pl.ds(0, 16)] = st0
    cursor[pl.ds(16, 16)] = st1

    scan_edges(place_step, with_ew=True)

    def pass_body(p, _):
        def zcp(j, _):
            pltpu.async_copy(zb, chunk.at[pl.ds(t * _WPT + j * _ZB, _ZB)],
                             sem)
            return 0
        lax.fori_loop(0, _WPT // _ZB, zcp, 0)

        @pl.when(last)
        def _():
            pltpu.async_copy(zb.at[pl.ds(0, 128)],
                             chunk.at[pl.ds(_CH, 128)], sem)

        def zdrain(j, _):
            pltpu.make_async_copy(zeros_hbm, zb, sem).wait()
            return 0
        lax.fori_loop(0, _WPT // _ZB, zdrain, 0)

        @pl.when(last)
        def _():
            pltpu.make_async_copy(zeros_hbm.at[pl.ds(0, 128)],
                                  zb.at[pl.ds(0, 128)], sem).wait()
        plsc.subcore_barrier()

        sv0 = start16[pl.ds(0, 16)]
        sv1 = start16[pl.ds(16, 16)]
        cv0 = cnt16[pl.ds(0, 16)]
        cv1 = cnt16[pl.ds(16, 16)]
        start_p = jnp.sum(jnp.where(lane == p, sv0, 0)
                          + jnp.where(lane == p - 16, sv1, 0))
        cnt_p = jnp.sum(jnp.where(lane == p, cv0, 0)
                        + jnp.where(lane == p - 16, cv1, 0))
        nsteps = (cnt_p + 15) >> 4

        def sc_body(k, _):
            @pl.when(k >= 16)
            def _():
                pltpu.make_async_copy(zeros_hbm.at[pl.ds(0, 16)], drain,
                                      sem).wait()
            off = start_p + k * 16
            idx16 = binidx[pl.ds(off, 16)]
            val16 = binval[pl.ds(off, 16)]
            m = (k * 16 + lane) < cnt_p
            idx16 = jnp.where(m, idx16, _DUMP)
            val16 = jnp.where(m, val16, 0.0)
            so = (k & 15) * 16
            stg[pl.ds(so, 16)] = val16
            pltpu.async_copy(stg.at[pl.ds(so, 16)], chunk.at[idx16], sem,
                             add=True)
            return 0
        lax.fori_loop(0, nsteps, sc_body, 0)

        def drain_body(j, _):
            pltpu.make_async_copy(zeros_hbm.at[pl.ds(0, 16)], drain,
                                  sem).wait()
            return 0
        lax.fori_loop(0, jnp.minimum(nsteps, 16), drain_body, 0)

        plsc.subcore_barrier()
        hbm_off = p * _CH + t * _WPT

        @pl.when(is_f2c)
        def _():
            pltpu.sync_copy(chunk.at[pl.ds(t * _WPT, _WPT)],
                            af_hbm.at[pl.ds(hbm_off, _WPT)])

        @pl.when(jnp.logical_not(is_f2c))
        def _():
            pltpu.sync_copy(chunk.at[pl.ds(t * _WPT, _WPT)],
                            ac_hbm.at[pl.ds(hbm_off, _WPT)])
        plsc.subcore_barrier()
        return 0
    lax.fori_loop(0, _NPASS, pass_body, 0)

    hs = _HB // _NS
    pltpu.sync_copy(hist, hstage.at[pl.ds(t * _HB, _HB)])
    plsc.subcore_barrier()
    pltpu.sync_copy(hstage.at[pl.ds(t * hs, hs)], acc)

    def red_body(j, _):
        pltpu.sync_copy(hstage.at[pl.ds(j * _HB + t * hs, hs)], tmp)

        def add_body(i, _):
            acc[pl.ds(i * 16, 16)] = acc[pl.ds(i * 16, 16)] + tmp[pl.ds(i * 16, 16)]
            return 0
        lax.fori_loop(0, hs // 16, add_body, 0)
        return 0
    lax.fori_loop(1, _NS, red_body, 0)

    @pl.when(is_f2c)
    def _():
        pltpu.sync_copy(acc, degc_hbm.at[pl.ds(t * hs, hs)])

    @pl.when(jnp.logical_not(is_f2c))
    def _():
        pltpu.sync_copy(acc, degf_hbm.at[pl.ds(t * hs, hs)])


def _densify(src, dst, ew_f2c, ew_c2f):
    mesh = plsc.VectorSubcoreMesh(core_axis_name="c", subcore_axis_name="s")
    ews = jnp.concatenate([ew_f2c, ew_c2f])
    zeros = jnp.zeros((_ZB,), jnp.float32)
    kern = pl.kernel(
        _densify_body,
        out_type=[
            jax.ShapeDtypeStruct((_MPAD,), jnp.float32),
            jax.ShapeDtypeStruct((_MPAD,), jnp.float32),
            jax.ShapeDtypeStruct((_HB,), jnp.float32),
            jax.ShapeDtypeStruct((_HB,), jnp.float32),
        ],
        mesh=mesh,
        compiler_params=pltpu.CompilerParams(needs_layout_passes=False),
        scratch_types=[
            pltpu.VMEM((_SROUND,), jnp.int32),
            pltpu.VMEM((_SROUND,), jnp.int32),
            pltpu.VMEM((_SROUND,), jnp.float32),
            pltpu.VMEM((_EBUF + 16,), jnp.int32),
            pltpu.VMEM((_EBUF + 16,), jnp.float32),
            pltpu.VMEM((_HB,), jnp.float32),
            pltpu.VMEM((_ZB,), jnp.float32),
            pltpu.VMEM((32,), jnp.int32),
            pltpu.VMEM((32,), jnp.int32),
            pltpu.VMEM((32,), jnp.int32),
            pltpu.VMEM((256,), jnp.float32),
            pltpu.VMEM((16,), jnp.float32),
            pltpu.VMEM((_HB // _NS,), jnp.float32),
            pltpu.VMEM((_HB // _NS,), jnp.float32),
            pltpu.VMEM_SHARED((_CH + 128,), jnp.float32),
            pltpu.VMEM_SHARED((_HB * _NS,), jnp.float32),
            pltpu.SemaphoreType.DMA,
        ],
    )
    af, ac, degc, degf = kern(src, dst, ews, zeros)
    a_f2c = af.reshape(_MPAD // 128, 128)
    a_c2f = ac.reshape(_MPAD // 128, 128)
    return a_f2c, a_c2f, degc[:N_CELL], degf[:N_FEAT]


def kernel(feat_ids, cell_ids, bf, src_f2c, dst_f2c, ew_f2c, ew_c2f, params):
    p = params
    a_f2c, a_c2f, deg_cell, deg_feat = _densify(src_f2c, dst_f2c, ew_f2c,
                                                ew_c2f)

    bn = bf.shape[1]
    bf_pad = jnp.pad(bf, ((0, 0), (0, 128 - bn)))
    we_pad = jnp.pad(p['W_extra'], ((0, 128 - bn), (0, 0)))

    hf = _init_feat(p['embed_feat'], p['in_W1'], p['in_b1'], p['in_g1'],
                    p['in_beta1'])
    hc = _init_cell(p['embed_cell'], bf_pad, we_pad, p['b_extra'], p['in_W0'],
                    p['in_b0'], p['in_g0'], p['in_beta0'])

    hist = []
    for layer in range(2):
        c = p['conv'][layer]
        hf_p = jnp.pad(hf, ((0, _PF - N_FEAT), (0, 0)))
        hc_p = jnp.pad(hc, ((0, _PC - N_CELL), (0, 0)))
        new_hc = _conv(a_f2c, hf_p, hc, deg_cell, c['f2c_Ws'], c['f2c_Wn'],
                       c['f2c_b'], c['norm_cell_g'], c['norm_cell_b'],
                       bm=800, n_dst=N_CELL, n_src=_PF)
        new_hf = _conv(a_c2f, hc_p, hf, deg_feat, c['c2f_Ws'], c['c2f_Wn'],
                       c['c2f_b'], c['norm_feat_g'], c['norm_feat_b'],
                       bm=200, n_dst=N_FEAT, n_src=_PC)
        hf, hc = new_hf, new_hc
        hist.append(hc)

    return _readout(hist[0], hist[1], p['ro_W0'], p['ro_b0'], p['ro_W1'],
                    p['ro_b1'], p['calib_a'], p['calib_b'])
